# baseline (speedup 1.0000x reference)
import functools
import math

import numpy as np
import jax
import jax.numpy as jnp
from jax import lax
from jax.experimental import pallas as pl
from jax.experimental.pallas import tpu as pltpu
from jax.experimental.pallas import tpu_sc as plsc


def _pos_encoding(n_rows, embed_dim):
    P = np.zeros((n_rows, embed_dim), dtype=np.float32)
    X = np.arange(n_rows, dtype=np.float32).reshape(-1, 1) / np.power(
        10000.0, np.arange(0, embed_dim, 2, dtype=np.float32) / embed_dim)
    P[:, 0::2] = np.sin(X)
    P[:, 1::2] = np.cos(X)
    return jnp.asarray(P)


@functools.cache
def _build(B, D):
    info = plsc.get_sparse_core_info()
    NC, NS, L = info.num_cores, info.num_subcores, info.num_lanes
    NW = NC * NS
    assert B % (8 * NW) == 0 and D % L == 0
    b_per_w = B // NW
    n_groups = b_per_w // L
    n_chunks = D // L
    scale = float(math.sqrt(D))
    mesh = plsc.VectorSubcoreMesh(core_axis_name="c", subcore_axis_name="s")

    @functools.partial(
        pl.kernel,
        mesh=mesh,
        compiler_params=pltpu.CompilerParams(
            needs_layout_passes=False, use_tc_tiling_on_sc=True),
        out_type=jax.ShapeDtypeStruct((B,), jnp.float32),
        scratch_types=[
            pltpu.VMEM((b_per_w,), jnp.int32),
            pltpu.VMEM((b_per_w,), jnp.int32),
            pltpu.VMEM((b_per_w, 2 * D), jnp.float32),
            pltpu.VMEM((b_per_w, 2 * D), jnp.float32),
            pltpu.VMEM((b_per_w, D), jnp.float32),
            pltpu.VMEM((b_per_w,), jnp.float32),
            pltpu.VMEM((b_per_w * L,), jnp.float32),
            pltpu.SemaphoreType.DMA,
            pltpu.SemaphoreType.DMA,
            pltpu.SemaphoreType.DMA,
        ],
    )
    def k(user_hbm, item_hbm, utab_hbm, itab_hbm, pos_hbm, out_hbm,
          uidx_v, iidx_v,
          urow_v, irow_v, pos_v, out_v, acc_v, su, si, sp):
        wid = lax.axis_index("s") * NC + lax.axis_index("c")
        base = wid * b_per_w
        pltpu.sync_copy(user_hbm.at[pl.ds(base, b_per_w)], uidx_v)
        pltpu.sync_copy(item_hbm.at[pl.ds(base, b_per_w)], iidx_v)
        cp = pltpu.async_copy(pos_hbm.at[pl.ds(base, b_per_w)], pos_v, sp)
        copies = [cp]
        for c in range(n_groups):
            sl = pl.ds(c * L, L)
            uv = uidx_v[sl]
            iv = iidx_v[sl]
            copies.append(pltpu.async_copy(
                utab_hbm.at[lax.shift_right_logical(uv, 1)],
                urow_v.at[sl], su))
            copies.append(pltpu.async_copy(
                itab_hbm.at[lax.shift_right_logical(iv, 1)],
                irow_v.at[sl], si))
        for c in copies:
            c.wait()
        for c in range(n_groups):
            uvec = uidx_v[pl.ds(c * L, L)]
            ivec = iidx_v[pl.ds(c * L, L)]
            for k in range(L):
                b = c * L + k
                uoff = (uvec[k] & 1) * D
                ioff = (ivec[k] & 1) * D
                acc = jnp.zeros((L,), jnp.float32)
                for j in range(n_chunks):
                    u = urow_v[b, pl.ds(uoff + j * L, L)]
                    i = irow_v[b, pl.ds(ioff + j * L, L)]
                    p = pos_v[b, pl.ds(j * L, L)]
                    acc = acc + (u * scale + p) * (i * scale + p)
                acc_v[pl.ds(b * L, L)] = acc
        lanes = lax.iota(jnp.int32, L)
        for g in range(n_groups):
            base_ids = (lanes + g * L) * L
            res = jnp.zeros((L,), jnp.float32)
            for l in range(L):
                res = res + plsc.load_gather(acc_v, [base_ids + l])
            out_v[pl.ds(g * L, L)] = res
        pltpu.sync_copy(out_v, out_hbm.at[pl.ds(base, b_per_w)])

    return k


def kernel(user, item, user_table, item_table):
    B = user.shape[0]
    D = user_table.shape[1]
    pos = _pos_encoding(B, D)
    ut2 = user_table.reshape(-1, 2 * D)
    it2 = item_table.reshape(-1, 2 * D)
    return _build(B, D)(user, item, ut2, it2, pos)


# speedup vs baseline: 1.5894x; 1.5894x over previous
import functools
import math

import numpy as np
import jax
import jax.numpy as jnp
from jax import lax
from jax.experimental import pallas as pl
from jax.experimental.pallas import tpu as pltpu
from jax.experimental.pallas import tpu_sc as plsc


def _pos_encoding(n_rows, embed_dim):
    P = np.zeros((n_rows, embed_dim), dtype=np.float32)
    X = np.arange(n_rows, dtype=np.float32).reshape(-1, 1) / np.power(
        10000.0, np.arange(0, embed_dim, 2, dtype=np.float32) / embed_dim)
    P[:, 0::2] = np.sin(X)
    P[:, 1::2] = np.cos(X)
    return jnp.asarray(P)


@functools.cache
def _build(B, D):
    info = plsc.get_sparse_core_info()
    NC, NS, L = info.num_cores, info.num_subcores, info.num_lanes
    NW = NC * NS
    assert B % (8 * NW) == 0 and D % L == 0
    b_per_w = B // NW
    n_groups = b_per_w // L
    n_chunks = D // L
    scale = float(math.sqrt(D))
    mesh = plsc.VectorSubcoreMesh(core_axis_name="c", subcore_axis_name="s")

    @functools.partial(
        pl.kernel,
        mesh=mesh,
        compiler_params=pltpu.CompilerParams(
            needs_layout_passes=False, use_tc_tiling_on_sc=True),
        out_type=jax.ShapeDtypeStruct((B,), jnp.float32),
        scratch_types=[
            pltpu.VMEM((b_per_w,), jnp.int32),
            pltpu.VMEM((b_per_w,), jnp.int32),
            pltpu.VMEM((b_per_w, D), jnp.float32),
            pltpu.VMEM((b_per_w, D), jnp.float32),
            pltpu.VMEM((b_per_w, D), jnp.float32),
            pltpu.VMEM((b_per_w,), jnp.float32),
            pltpu.VMEM((b_per_w * L,), jnp.float32),
            pltpu.SemaphoreType.DMA,
            pltpu.SemaphoreType.DMA,
            pltpu.SemaphoreType.DMA,
        ],
    )
    def k(user_hbm, item_hbm, utab_hbm, itab_hbm, pos_hbm, out_hbm,
          uidx_v, iidx_v, urow_v, irow_v, pos_v, out_v, acc_v, su, si, sp):
        wid = lax.axis_index("s") * NC + lax.axis_index("c")
        base = wid * b_per_w
        pltpu.sync_copy(user_hbm.at[pl.ds(base, b_per_w)], uidx_v)
        pltpu.sync_copy(item_hbm.at[pl.ds(base, b_per_w)], iidx_v)
        copies = [pltpu.async_copy(
            pos_hbm.at[pl.ds(base, b_per_w)], pos_v, sp)]
        for c in range(n_groups):
            uvec = uidx_v[pl.ds(c * L, L)]
            ivec = iidx_v[pl.ds(c * L, L)]
            for t in range(L):
                b = c * L + t
                copies.append(pltpu.async_copy(
                    utab_hbm.at[pl.ds(uvec[t], 1)],
                    urow_v.at[pl.ds(b, 1)], su))
                copies.append(pltpu.async_copy(
                    itab_hbm.at[pl.ds(ivec[t], 1)],
                    irow_v.at[pl.ds(b, 1)], si))
        for cp in copies:
            cp.wait()
        for b in range(b_per_w):
            acc = jnp.zeros((L,), jnp.float32)
            for j in range(n_chunks):
                u = urow_v[b, pl.ds(j * L, L)]
                i = irow_v[b, pl.ds(j * L, L)]
                p = pos_v[b, pl.ds(j * L, L)]
                acc = acc + (u * scale + p) * (i * scale + p)
            acc_v[pl.ds(b * L, L)] = acc
        lanes = lax.iota(jnp.int32, L)
        for g in range(n_groups):
            base_ids = (lanes + g * L) * L
            res = jnp.zeros((L,), jnp.float32)
            for l in range(L):
                res = res + plsc.load_gather(acc_v, [base_ids + l])
            out_v[pl.ds(g * L, L)] = res
        pltpu.sync_copy(out_v, out_hbm.at[pl.ds(base, b_per_w)])

    return k


def kernel(user, item, user_table, item_table):
    B = user.shape[0]
    D = user_table.shape[1]
    pos = _pos_encoding(B, D)
    return _build(B, D)(user, item, user_table, item_table, pos)
